# baseline (speedup 1.0000x reference)
import functools

import jax
import jax.numpy as jnp
from jax import lax
from jax.experimental import pallas as pl
from jax.experimental.pallas import tpu as pltpu
from jax.experimental.pallas import tpu_sc as plsc

NUM_TASKS = 3
PROMPT_LEN = 20
HIDDEN = 4096
BATCH = 1024

NUM_CORES = 2
NUM_SUBCORES = 16
NUM_WORKERS = NUM_CORES * NUM_SUBCORES

B_PER_TILE = BATCH // NUM_WORKERS
FLIGHT = 8


def _sc_lookup(task_ids, table):
    mesh = plsc.VectorSubcoreMesh(core_axis_name="c", subcore_axis_name="s")

    @functools.partial(
        pl.kernel,
        out_type=jax.ShapeDtypeStruct((BATCH, PROMPT_LEN, HIDDEN), jnp.float32),
        mesh=mesh,
        scratch_types=[
            pltpu.VMEM((B_PER_TILE,), jnp.int32),
            pltpu.VMEM_SHARED((NUM_TASKS, PROMPT_LEN, HIDDEN), jnp.float32),
            pltpu.SemaphoreType.DMA,
        ],
    )
    def run(idx_hbm, table_hbm, out_hbm, idx_v, sh_table, sem):
        c = lax.axis_index("c")
        s = lax.axis_index("s")
        wid = s * NUM_CORES + c
        base = wid * B_PER_TILE
        pltpu.sync_copy(idx_hbm.at[pl.ds(base, B_PER_TILE)], idx_v)

        @pl.when(s == 0)
        def _():
            pltpu.sync_copy(table_hbm, sh_table)

        plsc.subcore_barrier()

        def wait_one():
            pltpu.make_async_copy(sh_table.at[0], out_hbm.at[base], sem).wait()

        inflight = 0
        for g in range(B_PER_TILE // 16):
            vec = idx_v[pl.ds(g * 16, 16)]
            for i in range(16):
                tid = vec[i]
                pltpu.async_copy(
                    sh_table.at[tid], out_hbm.at[base + g * 16 + i], sem)
                inflight += 1
                if inflight >= FLIGHT:
                    wait_one()
                    inflight -= 1
        for _ in range(inflight):
            wait_one()

    return run(task_ids, table)


def kernel(task_ids, prompt_embeddings):
    return _sc_lookup(task_ids.astype(jnp.int32), prompt_embeddings)
